# baseline (speedup 1.0000x reference)
import jax
import jax.numpy as jnp
from jax.experimental import pallas as pl

MARGIN = 0.25
GAMMA = 256.0
O_N = -MARGIN
DELTA_N = MARGIN

_BLK_C = 2048


def _circle_loss_block(labels_ref, x_ref, o_ref):
    j = pl.program_id(0)
    x = x_ref[...]
    cos = jnp.clip(x, -1.0, 1.0)
    alpha_n = jnp.maximum(cos - O_N, 0.0)
    neg = alpha_n * (cos - DELTA_N)
    col = jax.lax.broadcasted_iota(jnp.int32, x.shape, 1) + j * _BLK_C
    lab = labels_ref[...]
    out = jnp.where(col == lab, cos, neg)
    o_ref[...] = out * GAMMA


def kernel(cos_theta, labels):
    b, c = cos_theta.shape
    labels2d = labels.astype(jnp.int32).reshape(b, 1)
    grid = (pl.cdiv(c, _BLK_C),)
    return pl.pallas_call(
        _circle_loss_block,
        grid=grid,
        in_specs=[
            pl.BlockSpec((b, 1), lambda j: (0, 0)),
            pl.BlockSpec((b, _BLK_C), lambda j: (0, j)),
        ],
        out_specs=pl.BlockSpec((b, _BLK_C), lambda j: (0, j)),
        out_shape=jax.ShapeDtypeStruct((b, c), jnp.float32),
    )(labels2d, cos_theta)
